# baseline (speedup 1.0000x reference)
import jax
import jax.numpy as jnp
from jax import lax
from jax.experimental import pallas as pl
from jax.experimental.pallas import tpu as pltpu, tpu_sc as plsc

N = 50000
T = 10
H = 128
D = 64
E = 200000
R = 3

NC = 2
NS = 16
LANES = 16

CHUNK = 8448
NCHUNK = 6
NP_OUT = CHUNK * NCHUNK
STRIPE = CHUNK // NS
ACC_ROWS = CHUNK + 8
DUMMY_ROW = CHUNK

EPT = 6272
E_PAD = EPT * NC * NS
SENT = 0x3FFFFFFF

P = 64
P_SHIFT = 6
NP_LIST = EPT // P
NLIST = R * NCHUNK * NC * NS

ROWS_BLK = 2000
NBLK = N // ROWS_BLK

_MESH = plsc.VectorSubcoreMesh(core_axis_name="c", subcore_axis_name="s",
                               num_cores=NC, num_subcores=NS)
_SC_PARAMS = pltpu.CompilerParams(needs_layout_passes=False)


def _bucketize_body(d0, s0, d1, s1, d2, s2, srcb, dstb, counts,
                    dstv, srcv, sidx, didx, cntv):
  core = lax.axis_index("c")
  sub = lax.axis_index("s")
  w = core * NS + sub
  dsts = (d0, d1, d2)
  srcs = (s0, s1, s2)

  for r in range(R):
    pltpu.sync_copy(dsts[r].at[pl.ds(w * EPT, EPT)], dstv)
    pltpu.sync_copy(srcs[r].at[pl.ds(w * EPT, EPT)], srcv)
    for c in range(NCHUNK):
      lo = c * CHUNK

      def cbody(i, cur, _lo=lo):
        dv = dstv[pl.ds(i * LANES, LANES)]
        sv = srcv[pl.ds(i * LANES, LANES)]
        m = (dv >= _lo) & (dv < _lo + CHUNK)
        mi = m.astype(jnp.int32)
        inc = plsc.cumsum(mi)
        pos = cur + inc - mi
        row = lax.shift_right_logical(pos, P_SHIFT)
        col = lax.bitwise_and(pos, P - 1)
        plsc.store_scatter(sidx, [row, col], sv, mask=m)
        plsc.store_scatter(didx, [row, col], dv - _lo, mask=m)
        return cur + jnp.sum(mi)

      cnt = lax.fori_loop(0, EPT // LANES, cbody, jnp.int32(0))

      pad_end = lax.shift_left(
          lax.shift_right_logical(cnt + 255, 8), 8)
      ntail = lax.shift_right_logical(pad_end - cnt + LANES - 1, 4)

      def tbody(k, _, _cnt=cnt, _pad_end=pad_end):
        pos = _cnt + k * LANES + lax.iota(jnp.int32, LANES)
        mk = pos < _pad_end
        row = lax.shift_right_logical(pos, P_SHIFT)
        col = lax.bitwise_and(pos, P - 1)
        plsc.store_scatter(sidx, [row, col],
                           lax.bitwise_and(pos * 397, 16383), mask=mk)
        plsc.store_scatter(didx, [row, col],
                           DUMMY_ROW + lax.bitwise_and(pos, 7),
                           mask=mk)
        return 0

      lax.fori_loop(0, ntail, tbody, 0)

      cntv[...] = jnp.full((LANES,), 0, jnp.int32) + cnt
      lid = (r * NCHUNK + c) * NC * NS + w
      pltpu.sync_copy(sidx, srcb.at[lid])
      pltpu.sync_copy(didx, dstb.at[lid])
      pltpu.sync_copy(cntv, counts.at[lid])


_BUCKETIZE = pl.kernel(
    _bucketize_body,
    out_type=[
        jax.ShapeDtypeStruct((NLIST, NP_LIST, P), jnp.int32),
        jax.ShapeDtypeStruct((NLIST, NP_LIST, P), jnp.int32),
        jax.ShapeDtypeStruct((NLIST, LANES), jnp.int32),
    ],
    mesh=_MESH,
    compiler_params=_SC_PARAMS,
    scratch_types=[
        pltpu.VMEM((EPT,), jnp.int32),
        pltpu.VMEM((EPT,), jnp.int32),
        pltpu.VMEM((NP_LIST, P), jnp.int32),
        pltpu.VMEM((NP_LIST, P), jnp.int32),
        pltpu.VMEM((LANES,), jnp.int32),
    ],
)


def _sc_layer(hout):
  def body(y0, y1, y2, srcb, dstb, counts, xlp, out,
           sidx, didx, rows_a, rows_b, rows_c, rows_d, cntv, acc,
           gsem_a, gsem_b, gsem_c, gsem_d, ssem_a, ssem_b, ssem_c, ssem_d):
    rows = (rows_a, rows_b, rows_c, rows_d)
    gsems = (gsem_a, gsem_b, gsem_c, gsem_d)
    ssems = (ssem_a, ssem_b, ssem_c, ssem_d)
    core = lax.axis_index("c")
    sub = lax.axis_index("s")
    ys = (y0, y1, y2)

    for ci in range(NCHUNK // NC):
      chunk = (NCHUNK // NC) * core + ci
      lo = chunk * CHUNK
      pltpu.sync_copy(xlp.at[pl.ds(lo + sub * STRIPE, STRIPE)],
                      acc.at[pl.ds(sub * STRIPE, STRIPE)])
      plsc.subcore_barrier()

      for r in range(R):
        for wl in range(NC):
          w = sub + NS * wl
          lid = (r * NCHUNK) * NC * NS + chunk * NC * NS + w
          pltpu.sync_copy(counts.at[lid], cntv)
          pltpu.sync_copy(srcb.at[lid], sidx)
          pltpu.sync_copy(dstb.at[lid], didx)
          cnt = cntv[...][0]
          nquad = lax.shift_right_logical(cnt + 255, 8)

          def qbody(q, _, _y=ys[r]):
            gs = []
            for b in range(4):
              gs.append(pltpu.async_copy(_y.at[sidx.at[4 * q + b]],
                                         rows[b], gsems[b]))
            ss = []
            for b in range(4):
              gs[b].wait()
              ss.append(pltpu.async_copy(rows[b], acc.at[didx.at[4 * q + b]],
                                         ssems[b], add=True))
            for b in range(4):
              ss[b].wait()
            return 0

          lax.fori_loop(0, nquad, qbody, 0)

      plsc.subcore_barrier()
      pltpu.sync_copy(acc.at[pl.ds(sub * STRIPE, STRIPE)],
                      out.at[pl.ds(lo + sub * STRIPE, STRIPE)])
      plsc.subcore_barrier()

  return pl.kernel(
      body,
      out_type=jax.ShapeDtypeStruct((NP_OUT, hout), jnp.float32),
      mesh=_MESH,
      compiler_params=_SC_PARAMS,
      scratch_types=[
          pltpu.VMEM((NP_LIST, P), jnp.int32),
          pltpu.VMEM((NP_LIST, P), jnp.int32),
          pltpu.VMEM((P, hout), jnp.float32),
          pltpu.VMEM((P, hout), jnp.float32),
          pltpu.VMEM((P, hout), jnp.float32),
          pltpu.VMEM((P, hout), jnp.float32),
          pltpu.VMEM((LANES,), jnp.int32),
          pltpu.VMEM_SHARED((ACC_ROWS, hout), jnp.float32),
      ] + [pltpu.SemaphoreType.DMA] * 8,
  )


def _k1_body(ids_ref, nemb_ref, temb_ref, w_ref, xl_ref, y0_ref, y1_ref,
             y2_ref):
  ids = ids_ref[0, 0, :]
  oh = (ids[:, None] == lax.broadcasted_iota(jnp.int32, (ROWS_BLK, 16), 1))
  x0 = nemb_ref[...] + jnp.dot(oh.astype(jnp.float32), temb_ref[...],
                               preferred_element_type=jnp.float32)
  yc = jnp.dot(x0, w_ref[...], preferred_element_type=jnp.float32)
  xl_ref[...] = yc[:, 0:H]
  y0_ref[...] = yc[:, H:2 * H]
  y1_ref[...] = yc[:, 2 * H:3 * H]
  y2_ref[...] = yc[:, 3 * H:4 * H]


def _k2_body(msg_ref, w_ref, x1_ref, xl_ref, y0_ref, y1_ref, y2_ref):
  x1 = jnp.maximum(msg_ref[...], 0.0)
  x1_ref[...] = x1
  yc = jnp.dot(x1, w_ref[...], preferred_element_type=jnp.float32)
  xl_ref[...] = yc[:, 0:H]
  y0_ref[...] = yc[:, H:2 * H]
  y1_ref[...] = yc[:, 2 * H:3 * H]
  y2_ref[...] = yc[:, 3 * H:4 * H]


def _k3_body(x1_ref, msg_ref, g_ref, b_ref, w_ref, xl_ref, y0_ref, y1_ref,
             y2_ref):
  h = msg_ref[...]
  mu = jnp.mean(h, axis=1, keepdims=True)
  cc = h - mu
  var = jnp.mean(cc * cc, axis=1, keepdims=True)
  hn = cc * lax.rsqrt(var + 1e-5) * g_ref[...] + b_ref[...]
  x2 = x1_ref[...] + jnp.maximum(hn, 0.0)
  yc = jnp.dot(x2, w_ref[...], preferred_element_type=jnp.float32)
  z = jnp.zeros((ROWS_BLK, D), jnp.float32)
  xl_ref[...] = jnp.concatenate([yc[:, 0:D], z], axis=1)
  y0_ref[...] = jnp.concatenate([yc[:, D:2 * D], z], axis=1)
  y1_ref[...] = jnp.concatenate([yc[:, 2 * D:3 * D], z], axis=1)
  y2_ref[...] = jnp.concatenate([yc[:, 3 * D:4 * D], z], axis=1)


def _row_spec(cols):
  return pl.BlockSpec((ROWS_BLK, cols), lambda i: (i, 0))


_K1 = pl.pallas_call(
    _k1_body,
    grid=(NBLK,),
    in_specs=[
        pl.BlockSpec((1, 1, ROWS_BLK), lambda i: (i, 0, 0)),
        _row_spec(H),
        pl.BlockSpec((16, H), lambda i: (0, 0)),
        pl.BlockSpec((H, 4 * H), lambda i: (0, 0)),
    ],
    out_specs=[_row_spec(H), _row_spec(H), _row_spec(H), _row_spec(H)],
    out_shape=[
        jax.ShapeDtypeStruct((NP_OUT, H), jnp.float32),
        jax.ShapeDtypeStruct((N, H), jnp.float32),
        jax.ShapeDtypeStruct((N, H), jnp.float32),
        jax.ShapeDtypeStruct((N, H), jnp.float32),
    ],
)

_K2 = pl.pallas_call(
    _k2_body,
    grid=(NBLK,),
    in_specs=[
        _row_spec(H),
        pl.BlockSpec((H, 4 * H), lambda i: (0, 0)),
    ],
    out_specs=[_row_spec(H), _row_spec(H), _row_spec(H), _row_spec(H),
               _row_spec(H)],
    out_shape=[
        jax.ShapeDtypeStruct((N, H), jnp.float32),
        jax.ShapeDtypeStruct((NP_OUT, H), jnp.float32),
        jax.ShapeDtypeStruct((N, H), jnp.float32),
        jax.ShapeDtypeStruct((N, H), jnp.float32),
        jax.ShapeDtypeStruct((N, H), jnp.float32),
    ],
)

_K3 = pl.pallas_call(
    _k3_body,
    grid=(NBLK,),
    in_specs=[
        _row_spec(H),
        _row_spec(H),
        pl.BlockSpec((1, H), lambda i: (0, 0)),
        pl.BlockSpec((1, H), lambda i: (0, 0)),
        pl.BlockSpec((H, 4 * D), lambda i: (0, 0)),
    ],
    out_specs=[_row_spec(H), _row_spec(H), _row_spec(H), _row_spec(H)],
    out_shape=[
        jax.ShapeDtypeStruct((NP_OUT, H), jnp.float32),
        jax.ShapeDtypeStruct((N, H), jnp.float32),
        jax.ShapeDtypeStruct((N, H), jnp.float32),
        jax.ShapeDtypeStruct((N, H), jnp.float32),
    ],
)

_SC_H = _sc_layer(H)


def kernel(node_type_ids, edge_index_r0, edge_index_r1, edge_index_r2,
           node_emb, type_emb, W_in, L_in, W_res, L_res, ln_g, ln_b,
           W_out, L_out):
  ids3 = node_type_ids.reshape(NBLK, 1, ROWS_BLK)
  tpad = jnp.pad(type_emb, ((0, 16 - T), (0, 0)))
  wcat1 = jnp.concatenate([L_in, W_in[0], W_in[1], W_in[2]], axis=1)
  wcat2 = jnp.concatenate([L_res, W_res[0], W_res[1], W_res[2]], axis=1)
  wcat3 = jnp.concatenate([L_out, W_out[0], W_out[1], W_out[2]], axis=1)

  dpad = jnp.full((E_PAD - E,), SENT, jnp.int32)
  spad = jnp.zeros((E_PAD - E,), jnp.int32)
  eidx = []
  for e in (edge_index_r0, edge_index_r1, edge_index_r2):
    eidx.append(jnp.concatenate([e[0], dpad]))
    eidx.append(jnp.concatenate([e[1], spad]))

  srcb, dstb, counts = _BUCKETIZE(*eidx)

  xl1, y10, y11, y12 = _K1(ids3, node_emb, tpad, wcat1)
  msg1 = _SC_H(y10, y11, y12, srcb, dstb, counts, xl1)
  x1, xl2, y20, y21, y22 = _K2(msg1, wcat2)
  msg2 = _SC_H(y20, y21, y22, srcb, dstb, counts, xl2)
  xl3, y30, y31, y32 = _K3(x1, msg2, ln_g.reshape(1, H), ln_b.reshape(1, H),
                           wcat3)
  msg3 = _SC_H(y30, y31, y32, srcb, dstb, counts, xl3)
  return msg3[:N, :D]


# speedup vs baseline: 1.2696x; 1.2696x over previous
import jax
import jax.numpy as jnp
from jax import lax
from jax.experimental import pallas as pl
from jax.experimental.pallas import tpu as pltpu, tpu_sc as plsc

N = 50000
T = 10
H = 128
D = 64
E = 200000
R = 3

NC = 2
NS = 16
LANES = 16

CHUNK = 8448
NCHUNK = 6
NP_OUT = CHUNK * NCHUNK
STRIPE = CHUNK // NS
ACC_ROWS = CHUNK + 8
DUMMY_ROW = CHUNK

EPT = 6272
E_PAD = EPT * NC * NS
SENT = 0x3FFFFFFF

P = 128
P_SHIFT = 7
NP_LIST = EPT // P
NLIST = R * NCHUNK * NC * NS

ROWS_BLK = 2000
NBLK = N // ROWS_BLK

_MESH = plsc.VectorSubcoreMesh(core_axis_name="c", subcore_axis_name="s",
                               num_cores=NC, num_subcores=NS)
_SC_PARAMS = pltpu.CompilerParams(needs_layout_passes=False)


def _bucketize_body(d0, s0, d1, s1, d2, s2, srcb, dstb, counts,
                    dstv, srcv, sidx, didx, cntv):
  core = lax.axis_index("c")
  sub = lax.axis_index("s")
  w = core * NS + sub
  dsts = (d0, d1, d2)
  srcs = (s0, s1, s2)

  for r in range(R):
    pltpu.sync_copy(dsts[r].at[pl.ds(w * EPT, EPT)], dstv)
    pltpu.sync_copy(srcs[r].at[pl.ds(w * EPT, EPT)], srcv)
    for c in range(NCHUNK):
      lo = c * CHUNK

      def cbody(i, cur, _lo=lo):
        dv = dstv[pl.ds(i * LANES, LANES)]
        sv = srcv[pl.ds(i * LANES, LANES)]
        m = (dv >= _lo) & (dv < _lo + CHUNK)
        mi = m.astype(jnp.int32)
        inc = plsc.cumsum(mi)
        pos = cur + inc - mi
        row = lax.shift_right_logical(pos, P_SHIFT)
        col = lax.bitwise_and(pos, P - 1)
        plsc.store_scatter(sidx, [row, col], sv, mask=m)
        plsc.store_scatter(didx, [row, col], dv - _lo, mask=m)
        return cur + jnp.sum(mi)

      cnt = lax.fori_loop(0, EPT // LANES, cbody, jnp.int32(0))

      pad_end = lax.shift_left(
          lax.shift_right_logical(cnt + 127, 7), 7)
      ntail = lax.shift_right_logical(pad_end - cnt + LANES - 1, 4)

      def tbody(k, _, _cnt=cnt, _pad_end=pad_end):
        pos = _cnt + k * LANES + lax.iota(jnp.int32, LANES)
        mk = pos < _pad_end
        row = lax.shift_right_logical(pos, P_SHIFT)
        col = lax.bitwise_and(pos, P - 1)
        plsc.store_scatter(sidx, [row, col],
                           lax.bitwise_and(pos * 397, 16383), mask=mk)
        plsc.store_scatter(didx, [row, col],
                           DUMMY_ROW + lax.bitwise_and(pos, 7),
                           mask=mk)
        return 0

      lax.fori_loop(0, ntail, tbody, 0)

      cntv[...] = jnp.full((LANES,), 0, jnp.int32) + cnt
      lid = (r * NCHUNK + c) * NC * NS + w
      pltpu.sync_copy(sidx, srcb.at[lid])
      pltpu.sync_copy(didx, dstb.at[lid])
      pltpu.sync_copy(cntv, counts.at[lid])


_BUCKETIZE = pl.kernel(
    _bucketize_body,
    out_type=[
        jax.ShapeDtypeStruct((NLIST, NP_LIST, P), jnp.int32),
        jax.ShapeDtypeStruct((NLIST, NP_LIST, P), jnp.int32),
        jax.ShapeDtypeStruct((NLIST, LANES), jnp.int32),
    ],
    mesh=_MESH,
    compiler_params=_SC_PARAMS,
    scratch_types=[
        pltpu.VMEM((EPT,), jnp.int32),
        pltpu.VMEM((EPT,), jnp.int32),
        pltpu.VMEM((NP_LIST, P), jnp.int32),
        pltpu.VMEM((NP_LIST, P), jnp.int32),
        pltpu.VMEM((LANES,), jnp.int32),
    ],
)


def _sc_layer(hout):
  def body(y0, y1, y2, srcb, dstb, counts, xlp, out,
           sidx, didx, rows_a, rows_b, cntv, acc, gsem_a, gsem_b):
    rows = (rows_a, rows_b)
    gsems = (gsem_a, gsem_b)
    core = lax.axis_index("c")
    sub = lax.axis_index("s")
    ys = (y0, y1, y2)

    for ci in range(NCHUNK // NC):
      chunk = (NCHUNK // NC) * core + ci
      lo = chunk * CHUNK
      pltpu.sync_copy(xlp.at[pl.ds(lo + sub * STRIPE, STRIPE)],
                      acc.at[pl.ds(sub * STRIPE, STRIPE)])
      plsc.subcore_barrier()

      for r in range(R):
        for wl in range(NC):
          w = sub + NS * wl
          lid = (r * NCHUNK) * NC * NS + chunk * NC * NS + w
          pltpu.sync_copy(counts.at[lid], cntv)
          pltpu.sync_copy(srcb.at[lid], sidx)
          pltpu.sync_copy(dstb.at[lid], didx)
          cnt = cntv[...][0]
          npieces = lax.shift_right_logical(cnt + 127, 7)
          npair = lax.shift_right_logical(npieces, 1)
          odd = lax.bitwise_and(npieces, 1)
          _y = ys[r]

          @pl.when(npieces > 0)
          def _():
            pltpu.async_copy(_y.at[sidx.at[0]], rows[0], gsems[0])

          def pbody(p, _, _y=_y):
            j0 = 2 * p
            j1 = 2 * p + 1
            pltpu.async_copy(_y.at[sidx.at[j1]], rows[1], gsems[1])
            pltpu.make_async_copy(_y.at[sidx.at[j0]], rows[0],
                                  gsems[0]).wait()
            pltpu.sync_copy(rows[0], acc.at[didx.at[j0]], add=True)
            jn = jnp.minimum(j1 + 1, npieces - 1)
            pltpu.async_copy(_y.at[sidx.at[jn]], rows[0], gsems[0])
            pltpu.make_async_copy(_y.at[sidx.at[j1]], rows[1],
                                  gsems[1]).wait()
            pltpu.sync_copy(rows[1], acc.at[didx.at[j1]], add=True)
            return 0

          lax.fori_loop(0, npair, pbody, 0)

          @pl.when(npieces > 0)
          def _():
            pltpu.make_async_copy(_y.at[sidx.at[0]], rows[0],
                                  gsems[0]).wait()

            @pl.when(odd == 1)
            def _():
              pltpu.sync_copy(rows[0], acc.at[didx.at[npieces - 1]],
                              add=True)

      plsc.subcore_barrier()
      pltpu.sync_copy(acc.at[pl.ds(sub * STRIPE, STRIPE)],
                      out.at[pl.ds(lo + sub * STRIPE, STRIPE)])
      plsc.subcore_barrier()

  return pl.kernel(
      body,
      out_type=jax.ShapeDtypeStruct((NP_OUT, hout), jnp.float32),
      mesh=_MESH,
      compiler_params=_SC_PARAMS,
      scratch_types=[
          pltpu.VMEM((NP_LIST, P), jnp.int32),
          pltpu.VMEM((NP_LIST, P), jnp.int32),
          pltpu.VMEM((P, hout), jnp.float32),
          pltpu.VMEM((P, hout), jnp.float32),
          pltpu.VMEM((LANES,), jnp.int32),
          pltpu.VMEM_SHARED((ACC_ROWS, hout), jnp.float32),
      ] + [pltpu.SemaphoreType.DMA] * 2,
  )


def _k1_body(ids_ref, nemb_ref, temb_ref, w_ref, xl_ref, y0_ref, y1_ref,
             y2_ref):
  ids = ids_ref[0, 0, :]
  oh = (ids[:, None] == lax.broadcasted_iota(jnp.int32, (ROWS_BLK, 16), 1))
  x0 = nemb_ref[...] + jnp.dot(oh.astype(jnp.float32), temb_ref[...],
                               preferred_element_type=jnp.float32)
  yc = jnp.dot(x0, w_ref[...], preferred_element_type=jnp.float32)
  xl_ref[...] = yc[:, 0:H]
  y0_ref[...] = yc[:, H:2 * H]
  y1_ref[...] = yc[:, 2 * H:3 * H]
  y2_ref[...] = yc[:, 3 * H:4 * H]


def _k2_body(msg_ref, w_ref, x1_ref, xl_ref, y0_ref, y1_ref, y2_ref):
  x1 = jnp.maximum(msg_ref[...], 0.0)
  x1_ref[...] = x1
  yc = jnp.dot(x1, w_ref[...], preferred_element_type=jnp.float32)
  xl_ref[...] = yc[:, 0:H]
  y0_ref[...] = yc[:, H:2 * H]
  y1_ref[...] = yc[:, 2 * H:3 * H]
  y2_ref[...] = yc[:, 3 * H:4 * H]


def _k3_body(x1_ref, msg_ref, g_ref, b_ref, w_ref, xl_ref, y0_ref, y1_ref,
             y2_ref):
  h = msg_ref[...]
  mu = jnp.mean(h, axis=1, keepdims=True)
  cc = h - mu
  var = jnp.mean(cc * cc, axis=1, keepdims=True)
  hn = cc * lax.rsqrt(var + 1e-5) * g_ref[...] + b_ref[...]
  x2 = x1_ref[...] + jnp.maximum(hn, 0.0)
  yc = jnp.dot(x2, w_ref[...], preferred_element_type=jnp.float32)
  z = jnp.zeros((ROWS_BLK, D), jnp.float32)
  xl_ref[...] = jnp.concatenate([yc[:, 0:D], z], axis=1)
  y0_ref[...] = jnp.concatenate([yc[:, D:2 * D], z], axis=1)
  y1_ref[...] = jnp.concatenate([yc[:, 2 * D:3 * D], z], axis=1)
  y2_ref[...] = jnp.concatenate([yc[:, 3 * D:4 * D], z], axis=1)


def _row_spec(cols):
  return pl.BlockSpec((ROWS_BLK, cols), lambda i: (i, 0))


_K1 = pl.pallas_call(
    _k1_body,
    grid=(NBLK,),
    in_specs=[
        pl.BlockSpec((1, 1, ROWS_BLK), lambda i: (i, 0, 0)),
        _row_spec(H),
        pl.BlockSpec((16, H), lambda i: (0, 0)),
        pl.BlockSpec((H, 4 * H), lambda i: (0, 0)),
    ],
    out_specs=[_row_spec(H), _row_spec(H), _row_spec(H), _row_spec(H)],
    out_shape=[
        jax.ShapeDtypeStruct((NP_OUT, H), jnp.float32),
        jax.ShapeDtypeStruct((N, H), jnp.float32),
        jax.ShapeDtypeStruct((N, H), jnp.float32),
        jax.ShapeDtypeStruct((N, H), jnp.float32),
    ],
)

_K2 = pl.pallas_call(
    _k2_body,
    grid=(NBLK,),
    in_specs=[
        _row_spec(H),
        pl.BlockSpec((H, 4 * H), lambda i: (0, 0)),
    ],
    out_specs=[_row_spec(H), _row_spec(H), _row_spec(H), _row_spec(H),
               _row_spec(H)],
    out_shape=[
        jax.ShapeDtypeStruct((N, H), jnp.float32),
        jax.ShapeDtypeStruct((NP_OUT, H), jnp.float32),
        jax.ShapeDtypeStruct((N, H), jnp.float32),
        jax.ShapeDtypeStruct((N, H), jnp.float32),
        jax.ShapeDtypeStruct((N, H), jnp.float32),
    ],
)

_K3 = pl.pallas_call(
    _k3_body,
    grid=(NBLK,),
    in_specs=[
        _row_spec(H),
        _row_spec(H),
        pl.BlockSpec((1, H), lambda i: (0, 0)),
        pl.BlockSpec((1, H), lambda i: (0, 0)),
        pl.BlockSpec((H, 4 * D), lambda i: (0, 0)),
    ],
    out_specs=[_row_spec(H), _row_spec(H), _row_spec(H), _row_spec(H)],
    out_shape=[
        jax.ShapeDtypeStruct((NP_OUT, H), jnp.float32),
        jax.ShapeDtypeStruct((N, H), jnp.float32),
        jax.ShapeDtypeStruct((N, H), jnp.float32),
        jax.ShapeDtypeStruct((N, H), jnp.float32),
    ],
)

_SC_H = _sc_layer(H)


def kernel(node_type_ids, edge_index_r0, edge_index_r1, edge_index_r2,
           node_emb, type_emb, W_in, L_in, W_res, L_res, ln_g, ln_b,
           W_out, L_out):
  ids3 = node_type_ids.reshape(NBLK, 1, ROWS_BLK)
  tpad = jnp.pad(type_emb, ((0, 16 - T), (0, 0)))
  wcat1 = jnp.concatenate([L_in, W_in[0], W_in[1], W_in[2]], axis=1)
  wcat2 = jnp.concatenate([L_res, W_res[0], W_res[1], W_res[2]], axis=1)
  wcat3 = jnp.concatenate([L_out, W_out[0], W_out[1], W_out[2]], axis=1)

  dpad = jnp.full((E_PAD - E,), SENT, jnp.int32)
  spad = jnp.zeros((E_PAD - E,), jnp.int32)
  eidx = []
  for e in (edge_index_r0, edge_index_r1, edge_index_r2):
    eidx.append(jnp.concatenate([e[0], dpad]))
    eidx.append(jnp.concatenate([e[1], spad]))

  srcb, dstb, counts = _BUCKETIZE(*eidx)

  xl1, y10, y11, y12 = _K1(ids3, node_emb, tpad, wcat1)
  msg1 = _SC_H(y10, y11, y12, srcb, dstb, counts, xl1)
  x1, xl2, y20, y21, y22 = _K2(msg1, wcat2)
  msg2 = _SC_H(y20, y21, y22, srcb, dstb, counts, xl2)
  xl3, y30, y31, y32 = _K3(x1, msg2, ln_g.reshape(1, H), ln_b.reshape(1, H),
                           wcat3)
  msg3 = _SC_H(y30, y31, y32, srcb, dstb, counts, xl3)
  return msg3[:N, :D]


# speedup vs baseline: 1.2743x; 1.0036x over previous
import jax
import jax.numpy as jnp
from jax import lax
from jax.experimental import pallas as pl
from jax.experimental.pallas import tpu as pltpu, tpu_sc as plsc

N = 50000
T = 10
H = 128
D = 64
E = 200000
R = 3

NC = 2
NS = 16
LANES = 16

CHUNK = 8448
NCHUNK = 6
NP_OUT = CHUNK * NCHUNK
STRIPE = CHUNK // NS
ACC_ROWS = CHUNK + 8
DUMMY_ROW = CHUNK

EPT = 6272
E_PAD = EPT * NC * NS
SENT = 0x3FFFFFFF

P = 128
P_SHIFT = 7
NP_LIST = EPT // P
NLIST = R * NCHUNK * NC * NS

ROWS_BLK = 2000
NBLK = N // ROWS_BLK

_MESH = plsc.VectorSubcoreMesh(core_axis_name="c", subcore_axis_name="s",
                               num_cores=NC, num_subcores=NS)
_SC_PARAMS = pltpu.CompilerParams(needs_layout_passes=False)


def _bucketize_body(d0, s0, d1, s1, d2, s2, srcb, dstb, counts,
                    dstv, srcv, sidx, didx, cntv):
  core = lax.axis_index("c")
  sub = lax.axis_index("s")
  w = core * NS + sub
  dsts = (d0, d1, d2)
  srcs = (s0, s1, s2)

  for r in range(R):
    pltpu.sync_copy(dsts[r].at[pl.ds(w * EPT, EPT)], dstv)
    pltpu.sync_copy(srcs[r].at[pl.ds(w * EPT, EPT)], srcv)
    for c in range(NCHUNK):
      lo = c * CHUNK

      def cbody(i, cur, _lo=lo):
        dv = dstv[pl.ds(i * LANES, LANES)]
        sv = srcv[pl.ds(i * LANES, LANES)]
        m = (dv >= _lo) & (dv < _lo + CHUNK)
        mi = m.astype(jnp.int32)
        inc = plsc.cumsum(mi)
        pos = cur + inc - mi
        row = lax.shift_right_logical(pos, P_SHIFT)
        col = lax.bitwise_and(pos, P - 1)
        plsc.store_scatter(sidx, [row, col], sv, mask=m)
        plsc.store_scatter(didx, [row, col], dv - _lo, mask=m)
        return cur + jnp.sum(mi)

      cnt = lax.fori_loop(0, EPT // LANES, cbody, jnp.int32(0),
                          unroll=2)

      pad_end = lax.shift_left(
          lax.shift_right_logical(cnt + 127, 7), 7)
      ntail = lax.shift_right_logical(pad_end - cnt + LANES - 1, 4)

      def tbody(k, _, _cnt=cnt, _pad_end=pad_end):
        pos = _cnt + k * LANES + lax.iota(jnp.int32, LANES)
        mk = pos < _pad_end
        row = lax.shift_right_logical(pos, P_SHIFT)
        col = lax.bitwise_and(pos, P - 1)
        plsc.store_scatter(sidx, [row, col],
                           lax.bitwise_and(pos * 397, 16383), mask=mk)
        plsc.store_scatter(didx, [row, col],
                           DUMMY_ROW + lax.bitwise_and(pos, 7),
                           mask=mk)
        return 0

      lax.fori_loop(0, ntail, tbody, 0)

      cntv[...] = jnp.full((LANES,), 0, jnp.int32) + cnt
      lid = (r * NCHUNK + c) * NC * NS + w
      pltpu.sync_copy(sidx, srcb.at[lid])
      pltpu.sync_copy(didx, dstb.at[lid])
      pltpu.sync_copy(cntv, counts.at[lid])


_BUCKETIZE = pl.kernel(
    _bucketize_body,
    out_type=[
        jax.ShapeDtypeStruct((NLIST, NP_LIST, P), jnp.int32),
        jax.ShapeDtypeStruct((NLIST, NP_LIST, P), jnp.int32),
        jax.ShapeDtypeStruct((NLIST, LANES), jnp.int32),
    ],
    mesh=_MESH,
    compiler_params=_SC_PARAMS,
    scratch_types=[
        pltpu.VMEM((EPT,), jnp.int32),
        pltpu.VMEM((EPT,), jnp.int32),
        pltpu.VMEM((NP_LIST, P), jnp.int32),
        pltpu.VMEM((NP_LIST, P), jnp.int32),
        pltpu.VMEM((LANES,), jnp.int32),
    ],
)


def _sc_layer(hout):
  def body(y0, y1, y2, srcb, dstb, counts, xlp, out,
           sidx, didx, rows_a, rows_b, cntv, acc, gsem_a, gsem_b):
    rows = (rows_a, rows_b)
    gsems = (gsem_a, gsem_b)
    core = lax.axis_index("c")
    sub = lax.axis_index("s")
    ys = (y0, y1, y2)

    for ci in range(NCHUNK // NC):
      chunk = (NCHUNK // NC) * core + ci
      lo = chunk * CHUNK
      pltpu.sync_copy(xlp.at[pl.ds(lo + sub * STRIPE, STRIPE)],
                      acc.at[pl.ds(sub * STRIPE, STRIPE)])
      plsc.subcore_barrier()

      for r in range(R):
        for wl in range(NC):
          w = sub + NS * wl
          lid = (r * NCHUNK) * NC * NS + chunk * NC * NS + w
          pltpu.sync_copy(counts.at[lid], cntv)
          pltpu.sync_copy(srcb.at[lid], sidx)
          pltpu.sync_copy(dstb.at[lid], didx)
          cnt = cntv[...][0]
          npieces = lax.shift_right_logical(cnt + 127, 7)
          npair = lax.shift_right_logical(npieces, 1)
          odd = lax.bitwise_and(npieces, 1)
          _y = ys[r]

          @pl.when(npieces > 0)
          def _():
            pltpu.async_copy(_y.at[sidx.at[0]], rows[0], gsems[0])

          def pbody(p, _, _y=_y):
            j0 = 2 * p
            j1 = 2 * p + 1
            pltpu.async_copy(_y.at[sidx.at[j1]], rows[1], gsems[1])
            pltpu.make_async_copy(_y.at[sidx.at[j0]], rows[0],
                                  gsems[0]).wait()
            pltpu.sync_copy(rows[0], acc.at[didx.at[j0]], add=True)
            jn = jnp.minimum(j1 + 1, npieces - 1)
            pltpu.async_copy(_y.at[sidx.at[jn]], rows[0], gsems[0])
            pltpu.make_async_copy(_y.at[sidx.at[j1]], rows[1],
                                  gsems[1]).wait()
            pltpu.sync_copy(rows[1], acc.at[didx.at[j1]], add=True)
            return 0

          lax.fori_loop(0, npair, pbody, 0)

          @pl.when(npieces > 0)
          def _():
            pltpu.make_async_copy(_y.at[sidx.at[0]], rows[0],
                                  gsems[0]).wait()

            @pl.when(odd == 1)
            def _():
              pltpu.sync_copy(rows[0], acc.at[didx.at[npieces - 1]],
                              add=True)

      plsc.subcore_barrier()
      pltpu.sync_copy(acc.at[pl.ds(sub * STRIPE, STRIPE)],
                      out.at[pl.ds(lo + sub * STRIPE, STRIPE)])
      plsc.subcore_barrier()

  return pl.kernel(
      body,
      out_type=jax.ShapeDtypeStruct((NP_OUT, hout), jnp.float32),
      mesh=_MESH,
      compiler_params=_SC_PARAMS,
      scratch_types=[
          pltpu.VMEM((NP_LIST, P), jnp.int32),
          pltpu.VMEM((NP_LIST, P), jnp.int32),
          pltpu.VMEM((P, hout), jnp.float32),
          pltpu.VMEM((P, hout), jnp.float32),
          pltpu.VMEM((LANES,), jnp.int32),
          pltpu.VMEM_SHARED((ACC_ROWS, hout), jnp.float32),
      ] + [pltpu.SemaphoreType.DMA] * 2,
  )


def _k1_body(ids_ref, nemb_ref, temb_ref, w_ref, xl_ref, y0_ref, y1_ref,
             y2_ref):
  ids = ids_ref[0, 0, :]
  oh = (ids[:, None] == lax.broadcasted_iota(jnp.int32, (ROWS_BLK, 16), 1))
  x0 = nemb_ref[...] + jnp.dot(oh.astype(jnp.float32), temb_ref[...],
                               preferred_element_type=jnp.float32)
  yc = jnp.dot(x0, w_ref[...], preferred_element_type=jnp.float32)
  xl_ref[...] = yc[:, 0:H]
  y0_ref[...] = yc[:, H:2 * H]
  y1_ref[...] = yc[:, 2 * H:3 * H]
  y2_ref[...] = yc[:, 3 * H:4 * H]


def _k2_body(msg_ref, w_ref, x1_ref, xl_ref, y0_ref, y1_ref, y2_ref):
  x1 = jnp.maximum(msg_ref[...], 0.0)
  x1_ref[...] = x1
  yc = jnp.dot(x1, w_ref[...], preferred_element_type=jnp.float32)
  xl_ref[...] = yc[:, 0:H]
  y0_ref[...] = yc[:, H:2 * H]
  y1_ref[...] = yc[:, 2 * H:3 * H]
  y2_ref[...] = yc[:, 3 * H:4 * H]


def _k3_body(x1_ref, msg_ref, g_ref, b_ref, w_ref, xl_ref, y0_ref, y1_ref,
             y2_ref):
  h = msg_ref[...]
  mu = jnp.mean(h, axis=1, keepdims=True)
  cc = h - mu
  var = jnp.mean(cc * cc, axis=1, keepdims=True)
  hn = cc * lax.rsqrt(var + 1e-5) * g_ref[...] + b_ref[...]
  x2 = x1_ref[...] + jnp.maximum(hn, 0.0)
  yc = jnp.dot(x2, w_ref[...], preferred_element_type=jnp.float32)
  z = jnp.zeros((ROWS_BLK, D), jnp.float32)
  xl_ref[...] = jnp.concatenate([yc[:, 0:D], z], axis=1)
  y0_ref[...] = jnp.concatenate([yc[:, D:2 * D], z], axis=1)
  y1_ref[...] = jnp.concatenate([yc[:, 2 * D:3 * D], z], axis=1)
  y2_ref[...] = jnp.concatenate([yc[:, 3 * D:4 * D], z], axis=1)


def _row_spec(cols):
  return pl.BlockSpec((ROWS_BLK, cols), lambda i: (i, 0))


_K1 = pl.pallas_call(
    _k1_body,
    grid=(NBLK,),
    in_specs=[
        pl.BlockSpec((1, 1, ROWS_BLK), lambda i: (i, 0, 0)),
        _row_spec(H),
        pl.BlockSpec((16, H), lambda i: (0, 0)),
        pl.BlockSpec((H, 4 * H), lambda i: (0, 0)),
    ],
    out_specs=[_row_spec(H), _row_spec(H), _row_spec(H), _row_spec(H)],
    out_shape=[
        jax.ShapeDtypeStruct((NP_OUT, H), jnp.float32),
        jax.ShapeDtypeStruct((N, H), jnp.float32),
        jax.ShapeDtypeStruct((N, H), jnp.float32),
        jax.ShapeDtypeStruct((N, H), jnp.float32),
    ],
)

_K2 = pl.pallas_call(
    _k2_body,
    grid=(NBLK,),
    in_specs=[
        _row_spec(H),
        pl.BlockSpec((H, 4 * H), lambda i: (0, 0)),
    ],
    out_specs=[_row_spec(H), _row_spec(H), _row_spec(H), _row_spec(H),
               _row_spec(H)],
    out_shape=[
        jax.ShapeDtypeStruct((N, H), jnp.float32),
        jax.ShapeDtypeStruct((NP_OUT, H), jnp.float32),
        jax.ShapeDtypeStruct((N, H), jnp.float32),
        jax.ShapeDtypeStruct((N, H), jnp.float32),
        jax.ShapeDtypeStruct((N, H), jnp.float32),
    ],
)

_K3 = pl.pallas_call(
    _k3_body,
    grid=(NBLK,),
    in_specs=[
        _row_spec(H),
        _row_spec(H),
        pl.BlockSpec((1, H), lambda i: (0, 0)),
        pl.BlockSpec((1, H), lambda i: (0, 0)),
        pl.BlockSpec((H, 4 * D), lambda i: (0, 0)),
    ],
    out_specs=[_row_spec(H), _row_spec(H), _row_spec(H), _row_spec(H)],
    out_shape=[
        jax.ShapeDtypeStruct((NP_OUT, H), jnp.float32),
        jax.ShapeDtypeStruct((N, H), jnp.float32),
        jax.ShapeDtypeStruct((N, H), jnp.float32),
        jax.ShapeDtypeStruct((N, H), jnp.float32),
    ],
)

_SC_H = _sc_layer(H)


def kernel(node_type_ids, edge_index_r0, edge_index_r1, edge_index_r2,
           node_emb, type_emb, W_in, L_in, W_res, L_res, ln_g, ln_b,
           W_out, L_out):
  ids3 = node_type_ids.reshape(NBLK, 1, ROWS_BLK)
  tpad = jnp.pad(type_emb, ((0, 16 - T), (0, 0)))
  wcat1 = jnp.concatenate([L_in, W_in[0], W_in[1], W_in[2]], axis=1)
  wcat2 = jnp.concatenate([L_res, W_res[0], W_res[1], W_res[2]], axis=1)
  wcat3 = jnp.concatenate([L_out, W_out[0], W_out[1], W_out[2]], axis=1)

  dpad = jnp.full((E_PAD - E,), SENT, jnp.int32)
  spad = jnp.zeros((E_PAD - E,), jnp.int32)
  eidx = []
  for e in (edge_index_r0, edge_index_r1, edge_index_r2):
    eidx.append(jnp.concatenate([e[0], dpad]))
    eidx.append(jnp.concatenate([e[1], spad]))

  srcb, dstb, counts = _BUCKETIZE(*eidx)

  xl1, y10, y11, y12 = _K1(ids3, node_emb, tpad, wcat1)
  msg1 = _SC_H(y10, y11, y12, srcb, dstb, counts, xl1)
  x1, xl2, y20, y21, y22 = _K2(msg1, wcat2)
  msg2 = _SC_H(y20, y21, y22, srcb, dstb, counts, xl2)
  xl3, y30, y31, y32 = _K3(x1, msg2, ln_g.reshape(1, H), ln_b.reshape(1, H),
                           wcat3)
  msg3 = _SC_H(y30, y31, y32, srcb, dstb, counts, xl3)
  return msg3[:N, :D]
